# Initial kernel scaffold; baseline (speedup 1.0000x reference)
import jax
import jax.numpy as jnp
from jax.experimental import pallas as pl


def kernel(x, w_gate, w1, w3, w2):
    raise NotImplementedError("write your pallas kernel here")


# speedup vs baseline: 1.4426x; 1.4426x over previous
import functools

import jax
import jax.numpy as jnp
from jax import lax
from jax.experimental import pallas as pl
from jax.experimental.pallas import tpu as pltpu

T, D, F, E, K = 512, 1024, 2048, 8, 2
M = 128
S = 2048
NT = S // M


def _routing_body(x_ref, wg_ref, p_ref, c_ref, te_ref, na_ref):
    x = x_ref[...]
    wg = wg_ref[...]

    lg = lax.dot_general(wg, x, (((1,), (1,)), ((), ())),
                         preferred_element_type=jnp.float32)
    mx = jnp.max(lg, axis=0, keepdims=True)
    ex = jnp.exp(lg - mx)
    probs = ex / jnp.sum(ex, axis=0, keepdims=True)

    io_e = lax.broadcasted_iota(jnp.int32, (E, T), 0)
    m1 = jnp.max(probs, axis=0, keepdims=True)
    e1 = jnp.min(jnp.where(probs == m1, io_e, E), axis=0, keepdims=True)
    oh1 = io_e == e1
    probs2 = jnp.where(oh1, -1.0, probs)
    m2 = jnp.max(probs2, axis=0, keepdims=True)
    e2 = jnp.min(jnp.where(probs2 == m2, io_e, E), axis=0, keepdims=True)
    oh2 = io_e == e2

    ssum = m1 + m2
    wa = m1 / ssum
    wb = m2 / ssum

    eq1 = oh1.astype(jnp.float32)
    eq2 = oh2.astype(jnp.float32)
    cnt1 = jnp.sum(eq1, axis=1, keepdims=True)
    cnt2 = jnp.sum(eq2, axis=1, keepdims=True)
    cnt = cnt1 + cnt2

    r = lax.broadcasted_iota(jnp.int32, (T, T), 0)
    c = lax.broadcasted_iota(jnp.int32, (T, T), 1)
    uincl = (r <= c).astype(jnp.float32)
    pc1 = lax.dot_general(eq1, uincl, (((1,), (0,)), ((), ())),
                          preferred_element_type=jnp.float32)
    pc2 = lax.dot_general(eq2, uincl, (((1,), (0,)), ((), ())),
                          preferred_element_type=jnp.float32)
    rank0 = jnp.sum(eq1 * pc1, axis=0, keepdims=True) - 1.0
    rank1 = jnp.sum(eq2 * (cnt1 + pc2), axis=0, keepdims=True) - 1.0

    cnti = cnt.astype(jnp.int32)
    cp = ((cnti + (M - 1)) // M) * M
    cpf = cp.astype(jnp.float32)
    re = lax.broadcasted_iota(jnp.int32, (E, E), 0)
    ce = lax.broadcasted_iota(jnp.int32, (E, E), 1)
    lstrict = (re > ce).astype(jnp.float32)
    offs = lax.dot_general(lstrict, cpf, (((1,), (0,)), ((), ())),
                           preferred_element_type=jnp.float32)

    pos0 = jnp.sum(eq1 * offs, axis=0, keepdims=True) + rank0
    pos1 = jnp.sum(eq2 * offs, axis=0, keepdims=True) + rank1
    pos0i = pos0.astype(jnp.int32)
    pos1i = pos1.astype(jnp.int32)

    io_s = lax.broadcasted_iota(jnp.int32, (S, T), 0)
    p0 = (io_s == pos0i).astype(jnp.float32)
    p1 = (io_s == pos1i).astype(jnp.float32)
    p_ref[...] = p0 + p1
    c_ref[...] = p0 * wa + p1 * wb

    total = jnp.sum(cpf)
    jm = (lax.broadcasted_iota(jnp.int32, (1, NT), 1)).astype(jnp.float32) * float(M)
    te = jnp.sum((offs <= jm).astype(jnp.int32), axis=0, keepdims=True) - 1
    active = jm < total
    te_last = jnp.max(jnp.where(active, te, -1), axis=1, keepdims=True)
    te_ref[...] = jnp.where(active, te, te_last)
    na_ref[...] = jnp.full((1, 1), 0, jnp.int32) + (
        total.astype(jnp.int32) // M)


def _gemm_body(te_ref, na_ref, p_ref, c_ref, x_ref, w1_ref, w3_ref, w2_ref,
               out_ref):
    j = pl.program_id(0)

    @pl.when(j == 0)
    def _init():
        out_ref[...] = jnp.zeros_like(out_ref)

    @pl.when(j < na_ref[0])
    def _work():
        pt = p_ref[...]
        xt = lax.dot_general(pt, x_ref[...], (((1,), (0,)), ((), ())),
                             preferred_element_type=jnp.float32)
        w1 = w1_ref[0]
        w3 = w3_ref[0]
        h1 = lax.dot_general(xt, w1, (((1,), (1,)), ((), ())),
                             preferred_element_type=jnp.float32)
        h3 = lax.dot_general(xt, w3, (((1,), (1,)), ((), ())),
                             preferred_element_type=jnp.float32)
        act = h1 * jax.nn.sigmoid(h1) * h3
        y = lax.dot_general(act, w2_ref[0], (((1,), (1,)), ((), ())),
                            preferred_element_type=jnp.float32)
        ct = c_ref[...]
        out_ref[...] += lax.dot_general(
            ct, y, (((0,), (0,)), ((), ())),
            preferred_element_type=jnp.float32)


@jax.jit
def kernel(x, w_gate, w1, w3, w2):
    p, c, te, na = pl.pallas_call(
        _routing_body,
        out_shape=[
            jax.ShapeDtypeStruct((S, T), jnp.float32),
            jax.ShapeDtypeStruct((S, T), jnp.float32),
            jax.ShapeDtypeStruct((1, NT), jnp.int32),
            jax.ShapeDtypeStruct((1, 1), jnp.int32),
        ],
    )(x, w_gate)

    te = te.reshape((NT,))
    na = na.reshape((1,))

    grid_spec = pltpu.PrefetchScalarGridSpec(
        num_scalar_prefetch=2,
        grid=(NT,),
        in_specs=[
            pl.BlockSpec((M, T), lambda j, te, na: (j, 0)),
            pl.BlockSpec((M, T), lambda j, te, na: (j, 0)),
            pl.BlockSpec((T, D), lambda j, te, na: (0, 0)),
            pl.BlockSpec((1, F, D), lambda j, te, na: (te[j], 0, 0)),
            pl.BlockSpec((1, F, D), lambda j, te, na: (te[j], 0, 0)),
            pl.BlockSpec((1, D, F), lambda j, te, na: (te[j], 0, 0)),
        ],
        out_specs=pl.BlockSpec((T, D), lambda j, te, na: (0, 0)),
    )
    out = pl.pallas_call(
        _gemm_body,
        grid_spec=grid_spec,
        out_shape=jax.ShapeDtypeStruct((T, D), jnp.float32),
    )(te, na, p, c, x, w1, w3, w2)
    return out


# speedup vs baseline: 1.8255x; 1.2654x over previous
import jax
import jax.numpy as jnp
from jax import lax
from jax.experimental import pallas as pl
from jax.experimental.pallas import tpu as pltpu

T, D, F, E, K = 512, 1024, 2048, 8, 2
M = 256
TMAX = (T + M - 1) // M


def _routing_body(x_ref, wg_ref, pos0_ref, pos1_ref, wa_ref, wb_ref,
                  offs_ref, ntl_ref):
    x = x_ref[...]
    wg = wg_ref[...]

    lg = lax.dot_general(wg, x, (((1,), (1,)), ((), ())),
                         preferred_element_type=jnp.float32)
    mx = jnp.max(lg, axis=0, keepdims=True)
    ex = jnp.exp(lg - mx)
    probs = ex / jnp.sum(ex, axis=0, keepdims=True)

    io_e = lax.broadcasted_iota(jnp.int32, (E, T), 0)
    m1 = jnp.max(probs, axis=0, keepdims=True)
    e1 = jnp.min(jnp.where(probs == m1, io_e, E), axis=0, keepdims=True)
    oh1 = io_e == e1
    probs2 = jnp.where(oh1, -1.0, probs)
    m2 = jnp.max(probs2, axis=0, keepdims=True)
    e2 = jnp.min(jnp.where(probs2 == m2, io_e, E), axis=0, keepdims=True)
    oh2 = io_e == e2

    ssum = m1 + m2
    wa_ref[...] = m1 / ssum
    wb_ref[...] = m2 / ssum

    eq1 = oh1.astype(jnp.float32)
    eq2 = oh2.astype(jnp.float32)
    cnt1 = jnp.sum(eq1, axis=1, keepdims=True)
    cnt2 = jnp.sum(eq2, axis=1, keepdims=True)
    cnt = cnt1 + cnt2

    r = lax.broadcasted_iota(jnp.int32, (T, T), 0)
    c = lax.broadcasted_iota(jnp.int32, (T, T), 1)
    uincl = (r <= c).astype(jnp.float32)
    pc1 = lax.dot_general(eq1, uincl, (((1,), (0,)), ((), ())),
                          preferred_element_type=jnp.float32)
    pc2 = lax.dot_general(eq2, uincl, (((1,), (0,)), ((), ())),
                          preferred_element_type=jnp.float32)
    rank0 = jnp.sum(eq1 * pc1, axis=0, keepdims=True) - 1.0
    rank1 = jnp.sum(eq2 * (cnt1 + pc2), axis=0, keepdims=True) - 1.0

    cnti = cnt.astype(jnp.int32)
    ntl = (cnti + (M - 1)) // M
    cpf = (ntl * M).astype(jnp.float32)
    re = lax.broadcasted_iota(jnp.int32, (E, E), 0)
    ce = lax.broadcasted_iota(jnp.int32, (E, E), 1)
    lstrict = (re > ce).astype(jnp.float32)
    offs = lax.dot_general(lstrict, cpf, (((1,), (0,)), ((), ())),
                           preferred_element_type=jnp.float32)

    pos0 = jnp.sum(eq1 * offs, axis=0, keepdims=True) + rank0
    pos1 = jnp.sum(eq2 * offs, axis=0, keepdims=True) + rank1
    pos0_ref[...] = pos0.astype(jnp.int32)
    pos1_ref[...] = pos1.astype(jnp.int32)

    offs_ref[...] = _transpose_col(offs.astype(jnp.int32))
    ntl_ref[...] = _transpose_col(ntl)


def _transpose_col(v):
    re = lax.broadcasted_iota(jnp.int32, (E, E), 0)
    ce = lax.broadcasted_iota(jnp.int32, (E, E), 1)
    return jnp.sum(jnp.where(re == ce, v, 0), axis=0, keepdims=True)


def _gemm_body(offs_ref, ntl_ref, pos0_ref, pos1_ref, wa_ref, wb_ref, x_ref,
               w1_ref, w3_ref, w2_ref, out_ref):
    e = pl.program_id(0)
    t = pl.program_id(1)
    tile = (TMAX - 1) - t

    @pl.when(jnp.logical_and(e == 0, t == 0))
    def _init():
        out_ref[...] = jnp.zeros_like(out_ref)

    @pl.when(tile < ntl_ref[e])
    def _work():
        base = offs_ref[e] + tile * M
        rows = lax.broadcasted_iota(jnp.int32, (M, T), 0) + base
        hit0 = rows == pos0_ref[...]
        hit1 = rows == pos1_ref[...]
        pt = hit0.astype(jnp.float32) + hit1.astype(jnp.float32)
        ct = (jnp.where(hit0, wa_ref[...], 0.0)
              + jnp.where(hit1, wb_ref[...], 0.0))
        xt = lax.dot_general(pt, x_ref[...], (((1,), (0,)), ((), ())),
                             preferred_element_type=jnp.float32)
        h1 = lax.dot_general(xt, w1_ref[0], (((1,), (1,)), ((), ())),
                             preferred_element_type=jnp.float32)
        h3 = lax.dot_general(xt, w3_ref[0], (((1,), (1,)), ((), ())),
                             preferred_element_type=jnp.float32)
        act = h1 * jax.nn.sigmoid(h1) * h3
        y = lax.dot_general(act, w2_ref[0], (((1,), (1,)), ((), ())),
                            preferred_element_type=jnp.float32)
        out_ref[...] += lax.dot_general(
            ct, y, (((0,), (0,)), ((), ())),
            preferred_element_type=jnp.float32)


@jax.jit
def kernel(x, w_gate, w1, w3, w2):
    pos0, pos1, wa, wb, offs, ntl = pl.pallas_call(
        _routing_body,
        out_shape=[
            jax.ShapeDtypeStruct((1, T), jnp.int32),
            jax.ShapeDtypeStruct((1, T), jnp.int32),
            jax.ShapeDtypeStruct((1, T), jnp.float32),
            jax.ShapeDtypeStruct((1, T), jnp.float32),
            jax.ShapeDtypeStruct((1, E), jnp.int32),
            jax.ShapeDtypeStruct((1, E), jnp.int32),
        ],
    )(x, w_gate)

    offs = offs.reshape((E,))
    ntl = ntl.reshape((E,))

    grid_spec = pltpu.PrefetchScalarGridSpec(
        num_scalar_prefetch=2,
        grid=(E, TMAX),
        in_specs=[
            pl.BlockSpec((1, T), lambda e, t, offs, ntl: (0, 0)),
            pl.BlockSpec((1, T), lambda e, t, offs, ntl: (0, 0)),
            pl.BlockSpec((1, T), lambda e, t, offs, ntl: (0, 0)),
            pl.BlockSpec((1, T), lambda e, t, offs, ntl: (0, 0)),
            pl.BlockSpec((T, D), lambda e, t, offs, ntl: (0, 0)),
            pl.BlockSpec((1, F, D), lambda e, t, offs, ntl: (e, 0, 0)),
            pl.BlockSpec((1, F, D), lambda e, t, offs, ntl: (e, 0, 0)),
            pl.BlockSpec((1, D, F), lambda e, t, offs, ntl: (e, 0, 0)),
        ],
        out_specs=pl.BlockSpec((T, D), lambda e, t, offs, ntl: (0, 0)),
    )
    out = pl.pallas_call(
        _gemm_body,
        grid_spec=grid_spec,
        out_shape=jax.ShapeDtypeStruct((T, D), jnp.float32),
    )(offs, ntl, pos0, pos1, wa, wb, x, w1, w3, w2)
    return out


# speedup vs baseline: 1.9175x; 1.0504x over previous
import jax
import jax.numpy as jnp
from jax import lax
from jax.experimental import pallas as pl
from jax.experimental.pallas import tpu as pltpu

T, D, F, E, K = 512, 1024, 2048, 8, 2
M = 256
TMAX = (T + M - 1) // M


def _routing(x, wg, pos0_s, pos1_s, wa_s, wb_s, offs_s, ntl_s):
    lg = lax.dot_general(wg, x, (((1,), (1,)), ((), ())),
                         preferred_element_type=jnp.float32)
    mx = jnp.max(lg, axis=0, keepdims=True)
    ex = jnp.exp(lg - mx)
    probs = ex / jnp.sum(ex, axis=0, keepdims=True)

    io_e = lax.broadcasted_iota(jnp.int32, (E, T), 0)
    m1 = jnp.max(probs, axis=0, keepdims=True)
    e1 = jnp.min(jnp.where(probs == m1, io_e, E), axis=0, keepdims=True)
    oh1 = io_e == e1
    probs2 = jnp.where(oh1, -1.0, probs)
    m2 = jnp.max(probs2, axis=0, keepdims=True)
    e2 = jnp.min(jnp.where(probs2 == m2, io_e, E), axis=0, keepdims=True)
    oh2 = io_e == e2

    ssum = m1 + m2
    wa_s[...] = m1 / ssum
    wb_s[...] = m2 / ssum

    eq1 = oh1.astype(jnp.float32)
    eq2 = oh2.astype(jnp.float32)
    cnt1 = jnp.sum(eq1, axis=1, keepdims=True)
    cnt2 = jnp.sum(eq2, axis=1, keepdims=True)
    cnt = cnt1 + cnt2

    r = lax.broadcasted_iota(jnp.int32, (T, T), 0)
    c = lax.broadcasted_iota(jnp.int32, (T, T), 1)
    uincl = (r <= c).astype(jnp.float32)
    pc1 = lax.dot_general(eq1, uincl, (((1,), (0,)), ((), ())),
                          preferred_element_type=jnp.float32)
    pc2 = lax.dot_general(eq2, uincl, (((1,), (0,)), ((), ())),
                          preferred_element_type=jnp.float32)
    rank0 = jnp.sum(eq1 * pc1, axis=0, keepdims=True) - 1.0
    rank1 = jnp.sum(eq2 * (cnt1 + pc2), axis=0, keepdims=True) - 1.0

    ntl = (cnt.astype(jnp.int32) + (M - 1)) // M
    cpf = (ntl * M).astype(jnp.float32)
    re = lax.broadcasted_iota(jnp.int32, (E, E), 0)
    ce = lax.broadcasted_iota(jnp.int32, (E, E), 1)
    lstrict = (re > ce).astype(jnp.float32)
    offs = lax.dot_general(lstrict, cpf, (((1,), (0,)), ((), ())),
                           preferred_element_type=jnp.float32)

    pos0_s[...] = (jnp.sum(eq1 * offs, axis=0, keepdims=True)
                   + rank0).astype(jnp.int32)
    pos1_s[...] = (jnp.sum(eq2 * offs, axis=0, keepdims=True)
                   + rank1).astype(jnp.int32)

    offs_i = offs.astype(jnp.int32)
    for ei in range(E):
        offs_s[ei] = offs_i[ei, 0]
        ntl_s[ei] = ntl[ei, 0]


def _body(x_ref, wg_ref, w1_ref, w3_ref, w2_ref, out_ref,
          pos0_s, pos1_s, wa_s, wb_s, offs_s, ntl_s):
    e = pl.program_id(0)
    t = pl.program_id(1)
    tile = (TMAX - 1) - t

    @pl.when(jnp.logical_and(e == 0, t == 0))
    def _init():
        out_ref[...] = jnp.zeros_like(out_ref)
        _routing(x_ref[...], wg_ref[...],
                 pos0_s, pos1_s, wa_s, wb_s, offs_s, ntl_s)

    @pl.when(tile < ntl_s[e])
    def _work():
        base = offs_s[e] + tile * M
        rows = lax.broadcasted_iota(jnp.int32, (M, T), 0) + base
        hit0 = rows == pos0_s[...]
        hit1 = rows == pos1_s[...]
        pt = hit0.astype(jnp.float32) + hit1.astype(jnp.float32)
        ct = (jnp.where(hit0, wa_s[...], 0.0)
              + jnp.where(hit1, wb_s[...], 0.0))
        xt = lax.dot_general(pt, x_ref[...], (((1,), (0,)), ((), ())),
                             preferred_element_type=jnp.float32)
        xb = xt.astype(jnp.bfloat16)
        w1 = w1_ref[0].astype(jnp.bfloat16)
        w3 = w3_ref[0].astype(jnp.bfloat16)
        h1 = lax.dot_general(xb, w1, (((1,), (1,)), ((), ())),
                             preferred_element_type=jnp.float32)
        h3 = lax.dot_general(xb, w3, (((1,), (1,)), ((), ())),
                             preferred_element_type=jnp.float32)
        act = h1 * jax.nn.sigmoid(h1) * h3
        y = lax.dot_general(act.astype(jnp.bfloat16),
                            w2_ref[0].astype(jnp.bfloat16),
                            (((1,), (1,)), ((), ())),
                            preferred_element_type=jnp.float32)
        out_ref[...] += lax.dot_general(
            ct, y, (((0,), (0,)), ((), ())),
            preferred_element_type=jnp.float32)


@jax.jit
def kernel(x, w_gate, w1, w3, w2):
    return pl.pallas_call(
        _body,
        grid=(E, TMAX),
        in_specs=[
            pl.BlockSpec((T, D), lambda e, t: (0, 0)),
            pl.BlockSpec((E, D), lambda e, t: (0, 0)),
            pl.BlockSpec((1, F, D), lambda e, t: (e, 0, 0)),
            pl.BlockSpec((1, F, D), lambda e, t: (e, 0, 0)),
            pl.BlockSpec((1, D, F), lambda e, t: (e, 0, 0)),
        ],
        out_specs=pl.BlockSpec((T, D), lambda e, t: (0, 0)),
        out_shape=jax.ShapeDtypeStruct((T, D), jnp.float32),
        scratch_shapes=[
            pltpu.VMEM((1, T), jnp.int32),
            pltpu.VMEM((1, T), jnp.int32),
            pltpu.VMEM((1, T), jnp.float32),
            pltpu.VMEM((1, T), jnp.float32),
            pltpu.SMEM((E,), jnp.int32),
            pltpu.SMEM((E,), jnp.int32),
        ],
    )(x, w_gate, w1, w3, w2)


# speedup vs baseline: 1.9633x; 1.0239x over previous
import jax
import jax.numpy as jnp
from jax import lax
from jax.experimental import pallas as pl
from jax.experimental.pallas import tpu as pltpu

T, D, F, E, K = 512, 1024, 2048, 8, 2
M = 256
TMAX = (T + M - 1) // M


def _routing(x, wg, pos0_s, pos1_s, wa_s, wb_s, offs_s, ntl_s):
    lg = lax.dot_general(wg, x, (((1,), (1,)), ((), ())),
                         preferred_element_type=jnp.float32)
    mx = jnp.max(lg, axis=0, keepdims=True)
    ex = jnp.exp(lg - mx)
    probs = ex / jnp.sum(ex, axis=0, keepdims=True)

    io_e = lax.broadcasted_iota(jnp.int32, (E, T), 0)
    m1 = jnp.max(probs, axis=0, keepdims=True)
    e1 = jnp.min(jnp.where(probs == m1, io_e, E), axis=0, keepdims=True)
    oh1 = io_e == e1
    probs2 = jnp.where(oh1, -1.0, probs)
    m2 = jnp.max(probs2, axis=0, keepdims=True)
    e2 = jnp.min(jnp.where(probs2 == m2, io_e, E), axis=0, keepdims=True)
    oh2 = io_e == e2

    ssum = m1 + m2
    wa_s[...] = m1 / ssum
    wb_s[...] = m2 / ssum

    eq1 = oh1.astype(jnp.float32)
    eq2 = oh2.astype(jnp.float32)
    cnt1 = jnp.sum(eq1, axis=1, keepdims=True)
    cnt2 = jnp.sum(eq2, axis=1, keepdims=True)
    cnt = cnt1 + cnt2

    r = lax.broadcasted_iota(jnp.int32, (T, T), 0)
    c = lax.broadcasted_iota(jnp.int32, (T, T), 1)
    uincl = (r <= c).astype(jnp.float32)
    pc1 = lax.dot_general(eq1, uincl, (((1,), (0,)), ((), ())),
                          preferred_element_type=jnp.float32)
    pc2 = lax.dot_general(eq2, uincl, (((1,), (0,)), ((), ())),
                          preferred_element_type=jnp.float32)
    rank0 = jnp.sum(eq1 * pc1, axis=0, keepdims=True) - 1.0
    rank1 = jnp.sum(eq2 * (cnt1 + pc2), axis=0, keepdims=True) - 1.0

    ntl = (cnt.astype(jnp.int32) + (M - 1)) // M
    cpf = (ntl * M).astype(jnp.float32)
    re = lax.broadcasted_iota(jnp.int32, (E, E), 0)
    ce = lax.broadcasted_iota(jnp.int32, (E, E), 1)
    lstrict = (re > ce).astype(jnp.float32)
    offs = lax.dot_general(lstrict, cpf, (((1,), (0,)), ((), ())),
                           preferred_element_type=jnp.float32)

    pos0_s[...] = (jnp.sum(eq1 * offs, axis=0, keepdims=True)
                   + rank0).astype(jnp.int32)
    pos1_s[...] = (jnp.sum(eq2 * offs, axis=0, keepdims=True)
                   + rank1).astype(jnp.int32)

    offs_i = offs.astype(jnp.int32)
    for ei in range(E):
        offs_s[ei] = offs_i[ei, 0]
        ntl_s[ei] = ntl[ei, 0]


def _body(x_ref, wg_ref, w1_ref, w3_ref, w2_ref, out_ref,
          pos0_s, pos1_s, wa_s, wb_s, offs_s, ntl_s):
    e = pl.program_id(0)
    t = pl.program_id(1)
    tile = (TMAX - 1) - t

    @pl.when(jnp.logical_and(e == 0, t == 0))
    def _init():
        out_ref[...] = jnp.zeros_like(out_ref)
        _routing(x_ref[...], wg_ref[...],
                 pos0_s, pos1_s, wa_s, wb_s, offs_s, ntl_s)

    @pl.when(tile < ntl_s[e])
    def _work():
        base = offs_s[e] + tile * M
        rows = lax.broadcasted_iota(jnp.int32, (M, T), 0) + base
        hit0 = rows == pos0_s[...]
        hit1 = rows == pos1_s[...]
        pt = hit0.astype(jnp.float32) + hit1.astype(jnp.float32)
        ct = (jnp.where(hit0, wa_s[...], 0.0)
              + jnp.where(hit1, wb_s[...], 0.0))
        xt = lax.dot_general(pt, x_ref[...], (((1,), (0,)), ((), ())),
                             preferred_element_type=jnp.float32)
        bf1 = lax.Precision.DEFAULT
        h1 = lax.dot_general(xt, w1_ref[0], (((1,), (1,)), ((), ())),
                             precision=bf1,
                             preferred_element_type=jnp.float32)
        h3 = lax.dot_general(xt, w3_ref[0], (((1,), (1,)), ((), ())),
                             precision=bf1,
                             preferred_element_type=jnp.float32)
        act = h1 * jax.nn.sigmoid(h1) * h3
        y = lax.dot_general(act, w2_ref[0], (((1,), (1,)), ((), ())),
                            precision=bf1,
                            preferred_element_type=jnp.float32)
        out_ref[...] += lax.dot_general(
            ct, y, (((0,), (0,)), ((), ())),
            preferred_element_type=jnp.float32)


@jax.jit
def kernel(x, w_gate, w1, w3, w2):
    return pl.pallas_call(
        _body,
        grid=(E, TMAX),
        in_specs=[
            pl.BlockSpec((T, D), lambda e, t: (0, 0)),
            pl.BlockSpec((E, D), lambda e, t: (0, 0)),
            pl.BlockSpec((1, F, D), lambda e, t: (e, 0, 0)),
            pl.BlockSpec((1, F, D), lambda e, t: (e, 0, 0)),
            pl.BlockSpec((1, D, F), lambda e, t: (e, 0, 0)),
        ],
        out_specs=pl.BlockSpec((T, D), lambda e, t: (0, 0)),
        out_shape=jax.ShapeDtypeStruct((T, D), jnp.float32),
        scratch_shapes=[
            pltpu.VMEM((1, T), jnp.int32),
            pltpu.VMEM((1, T), jnp.int32),
            pltpu.VMEM((1, T), jnp.float32),
            pltpu.VMEM((1, T), jnp.float32),
            pltpu.SMEM((E,), jnp.int32),
            pltpu.SMEM((E,), jnp.int32),
        ],
    )(x, w_gate, w1, w3, w2)
